# baseline (speedup 1.0000x reference)
import jax
import jax.numpy as jnp
from jax.experimental import pallas as pl


_BM = 400


def _linear_kernel(inp_ref, w_ref, out_ref):
    out_ref[...] = jnp.dot(inp_ref[...], w_ref[...],
                           preferred_element_type=jnp.float32)


def _spmm_kernel(adj_ref, s_ref, b_ref, out_ref):
    out_ref[...] = jnp.dot(adj_ref[...], s_ref[...],
                           preferred_element_type=jnp.float32) + b_ref[...]


def kernel(inp, adj_mat, kernel, bias):
    n, d_in = inp.shape
    d_out = kernel.shape[1]

    support = pl.pallas_call(
        _linear_kernel,
        out_shape=jax.ShapeDtypeStruct((n, d_out), jnp.float32),
    )(inp, kernel)

    out = pl.pallas_call(
        _spmm_kernel,
        grid=(n // _BM,),
        in_specs=[
            pl.BlockSpec((_BM, n), lambda i: (i, 0)),
            pl.BlockSpec((n, d_out), lambda i: (0, 0)),
            pl.BlockSpec((1, d_out), lambda i: (0, 0)),
        ],
        out_specs=pl.BlockSpec((_BM, d_out), lambda i: (i, 0)),
        out_shape=jax.ShapeDtypeStruct((n, d_out), jnp.float32),
    )(adj_mat, support, bias.reshape(1, d_out))
    return out


# speedup vs baseline: 1.0082x; 1.0082x over previous
import jax
import jax.numpy as jnp
from jax.experimental import pallas as pl


_BM = 400


def _linear_kernel(inp_ref, w_ref, out_ref):
    out_ref[...] = jnp.dot(inp_ref[...], w_ref[...],
                           preferred_element_type=jnp.float32)


def _spmm_kernel(adj_ref, s_ref, b_ref, out_ref):
    a = adj_ref[...].astype(jnp.bfloat16)
    s = s_ref[...].astype(jnp.bfloat16)
    out_ref[...] = jnp.dot(a, s,
                           preferred_element_type=jnp.float32) + b_ref[...]


def kernel(inp, adj_mat, kernel, bias):
    n, d_in = inp.shape
    d_out = kernel.shape[1]

    support = pl.pallas_call(
        _linear_kernel,
        out_shape=jax.ShapeDtypeStruct((n, d_out), jnp.float32),
    )(inp, kernel)

    out = pl.pallas_call(
        _spmm_kernel,
        grid=(n // _BM,),
        in_specs=[
            pl.BlockSpec((_BM, n), lambda i: (i, 0)),
            pl.BlockSpec((n, d_out), lambda i: (0, 0)),
            pl.BlockSpec((1, d_out), lambda i: (0, 0)),
        ],
        out_specs=pl.BlockSpec((_BM, d_out), lambda i: (i, 0)),
        out_shape=jax.ShapeDtypeStruct((n, d_out), jnp.float32),
    )(adj_mat, support, bias.reshape(1, d_out))
    return out


# speedup vs baseline: 1.0476x; 1.0390x over previous
import jax
import jax.numpy as jnp
from jax.experimental import pallas as pl
from jax.experimental.pallas import tpu as pltpu


_BM = 400


def _fused_kernel(adj_ref, inp_ref, w_ref, b_ref, out_ref, s_ref):
    @pl.when(pl.program_id(0) == 0)
    def _():
        s_ref[...] = jnp.dot(inp_ref[...], w_ref[...],
                             preferred_element_type=jnp.float32)

    out_ref[...] = jnp.dot(adj_ref[...], s_ref[...],
                           preferred_element_type=jnp.float32) + b_ref[...]


def kernel(inp, adj_mat, kernel, bias):
    n, d_in = inp.shape
    d_out = kernel.shape[1]

    out = pl.pallas_call(
        _fused_kernel,
        grid=(n // _BM,),
        in_specs=[
            pl.BlockSpec((_BM, n), lambda i: (i, 0)),
            pl.BlockSpec((n, d_in), lambda i: (0, 0)),
            pl.BlockSpec((d_in, d_out), lambda i: (0, 0)),
            pl.BlockSpec((1, d_out), lambda i: (0, 0)),
        ],
        out_specs=pl.BlockSpec((_BM, d_out), lambda i: (i, 0)),
        out_shape=jax.ShapeDtypeStruct((n, d_out), jnp.float32),
        scratch_shapes=[pltpu.VMEM((n, d_out), jnp.float32)],
    )(adj_mat, inp, kernel, bias.reshape(1, d_out))
    return out
